# baseline (speedup 1.0000x reference)
import functools

import jax
import jax.numpy as jnp
from jax import lax
from jax.experimental import pallas as pl
from jax.experimental.pallas import tpu as pltpu
from jax.experimental.pallas import tpu_sc as plsc

N = 5000
NPAD = 5056
C = 20
B = 2
MAXD = 300
MPAD = 304
PPAD = 320
NB = 128
NCH = 8
THR = 0.05
L = 16
NV = NPAD // L
BIGI = 1 << 30


def _maxl(xs):
    out = xs[0]
    for x in xs[1:]:
        out = jnp.maximum(out, x)
    return out


def _nms_body(boxes_hbm, scoresT_hbm, outb_hbm, outs_hbm, outl_hbm,
              sc0, ccs0, cci0, css0, cii0, p10, p20, p30, p40, oi0, os0,
              sc1, ccs1, cci1, css1, cii1, p11, p21, p31, p41, oi1, os1,
              boxes_v, sh_idx, sh_score, midx_v, mscore_v, pos_v,
              outb_v, outs_v, outl_v):
    b = lax.axis_index("c")
    sid = lax.axis_index("s")

    lane = lax.iota(jnp.int32, L)
    lane0 = lane == 0
    zf = jnp.zeros((L,), jnp.float32)
    neg = jnp.full((L,), -jnp.inf, jnp.float32)
    negi1 = jnp.full((L,), -1, jnp.int32)
    bigv = jnp.full((L,), BIGI, jnp.int32)
    thrv = jnp.full((L,), jnp.float32(THR))
    k0 = jnp.zeros((L,), jnp.int32)
    k1 = jnp.full((L,), 1, jnp.int32)
    k2 = jnp.full((L,), 2, jnp.int32)
    k3 = jnp.full((L,), 3, jnp.int32)

    pltpu.sync_copy(boxes_hbm.at[b], boxes_v)

    prob0 = dict(sc=sc0, ccs=ccs0, cci=cci0, css=css0, cii=cii0,
                 py1=p10, px1=p20, py2=p30, px2=p40, oi=oi0, os=os0)
    prob1 = dict(sc=sc1, ccs=ccs1, cci=cci1, css=css1, cii=cii1,
                 py1=p11, px1=p21, py2=p31, px2=p41, oi=oi1, os=os1)

    def process_multi(probs, clss):
        nP = len(probs)
        for P, cls in zip(probs, clss):
            pltpu.sync_copy(scoresT_hbm.at[b, cls], P["sc"])

        def init_body(v, carry):
            sl = pl.ds(v * L, L)
            for P in probs:
                P["py1"][sl] = zf
                P["px1"][sl] = zf
                P["py2"][sl] = zf
                P["px2"][sl] = zf
            return carry

        lax.fori_loop(0, PPAD // L, init_body, 0)

        def inito_body(v, carry):
            sl = pl.ds(v * L, L)
            for P in probs:
                P["oi"][sl] = negi1
                P["os"][sl] = neg
            return carry

        lax.fori_loop(0, MPAD // L, inito_body, 0)

        if nP > 1:
            def memset_body(v, carry):
                sl = pl.ds(v * L, L)
                for P in probs:
                    P["ccs"][sl] = neg
                return carry

            lax.fori_loop(0, (NPAD + 4 * L) // L, memset_body, 0)

            def memset2_body(v, carry):
                sl = pl.ds(v * L, L)
                for P in probs:
                    P["css"][sl] = neg
                return carry

            lax.fori_loop(0, (NPAD + L) // L, memset2_body, 0)

        def chunk_body(cidx, npks):
            actives = [npk < MAXD for npk in npks]
            any_active = actives[0]
            for a in actives[1:]:
                any_active = any_active | a
            activevs = [jnp.full((L,), a) for a in actives]
            klo = (NB - 16) - 16 * cidx
            lo = klo.astype(jnp.float32) * jnp.float32(1.0 / NB)
            hi = jnp.where(cidx == 0, jnp.float32(jnp.inf),
                           (klo + 16).astype(jnp.float32) * jnp.float32(1.0 / NB))
            lov = jnp.full((L,), lo)
            hiv = jnp.full((L,), hi)

            def cscan4(i4, cnts):
                cnts = list(cnts)
                for u in range(4):
                    off = (i4 * 4 + u) * L
                    iv = lane + off
                    for pi, P in enumerate(probs):
                        s = P["sc"][pl.ds(off, L)]
                        g = (s >= lov) & (s < hiv) & (s > thrv) & activevs[pi]
                        csum = plsc.cumsum(g.astype(jnp.int32))
                        pos = (cnts[pi] - 1) + csum
                        plsc.store_scatter(P["ccs"], [pos], s, mask=g)
                        plsc.store_scatter(P["cci"], [pos], iv, mask=g)
                        cnts[pi] = cnts[pi] + jnp.full((L,), csum[15])
                return tuple(cnts)

            ccntvs = lax.fori_loop(0, jnp.where(any_active, NV // 4, 0), cscan4,
                                   (jnp.zeros((L,), jnp.int32),) * nP)
            for pi, P in enumerate(probs):
                for u in range(4):
                    plsc.store_scatter(P["ccs"], [lane + ccntvs[pi] + u * L], neg)
            ccnts = [cv[15] for cv in ccntvs]
            nv4s = [lax.shift_right_logical(cc + 63, 6) for cc in ccnts]

            def bucket_body(b2, npks2):
                k = (klo + 15) - b2
                lives = [npk < MAXD for npk in npks2]
                livevs = [jnp.full((L,), a) for a in lives]
                blo = k.astype(jnp.float32) * jnp.float32(1.0 / NB)
                bhi = jnp.where(k == NB - 1, jnp.float32(jnp.inf),
                                (k + 1).astype(jnp.float32) * jnp.float32(1.0 / NB))
                blov = jnp.full((L,), blo)
                bhiv = jnp.full((L,), bhi)

                def bfilter4(i4, cnts):
                    cnts = list(cnts)
                    for u in range(4):
                        off = (i4 * 4 + u) * L
                        for pi, P in enumerate(probs):
                            s = P["ccs"][pl.ds(off, L)]
                            g = (s >= blov) & (s < bhiv) & livevs[pi]
                            iv = P["cci"][pl.ds(off, L)]
                            csum = plsc.cumsum(g.astype(jnp.int32))
                            pos = (cnts[pi] - 1) + csum
                            plsc.store_scatter(P["css"], [pos], s, mask=g)
                            plsc.store_scatter(P["cii"], [pos], iv, mask=g)
                            cnts[pi] = cnts[pi] + jnp.full((L,), csum[15])
                    return tuple(cnts)

                ftrip = _maxl([jnp.where(lv, n4, 0)
                               for lv, n4 in zip(lives, nv4s)])
                cntvs = lax.fori_loop(0, ftrip, bfilter4,
                                      (jnp.zeros((L,), jnp.int32),) * nP)
                for pi, P in enumerate(probs):
                    plsc.store_scatter(P["css"], [lane + cntvs[pi]], neg)
                cnts = [cv[15] for cv in cntvs]
                nvbs = [lax.shift_right_logical(cn + (L - 1), 4) for cn in cnts]

                def walk_body(t, npks3):
                    lives2 = [npk < MAXD for npk in npks3]
                    mptrips = [jnp.where(lv, nvb, 0)
                               for lv, nvb in zip(lives2, nvbs)]

                    def mp_body(v, st3):
                        st3 = list(st3)
                        posn = lane + v * L
                        for pi, P in enumerate(probs):
                            mv, pv = st3[2 * pi], st3[2 * pi + 1]
                            s = P["css"][pl.ds(v * L, L)]
                            upd = s > mv
                            st3[2 * pi] = jnp.where(upd, s, mv)
                            st3[2 * pi + 1] = jnp.where(upd, posn, pv)
                        return tuple(st3)

                    mpst = lax.fori_loop(0, _maxl(mptrips), mp_body,
                                         (neg, bigv) * nP)

                    acts, mms, bivs, cys = [], [], [], []
                    for pi, P in enumerate(probs):
                        mv, pv = mpst[2 * pi], mpst[2 * pi + 1]
                        m = jnp.max(mv)
                        act = lives2[pi] & (m > jnp.float32(-jnp.inf))
                        actv = jnp.full((L,), act)
                        mm = jnp.full((L,), m)
                        j = jnp.min(jnp.where(mv == mm, pv, bigv))
                        jv = jnp.full((L,), jnp.where(act, j, 0), jnp.int32)
                        biv = plsc.load_gather(P["cii"], [jv])
                        biv = jnp.where(actv, biv, k0)
                        plsc.store_scatter(P["css"], [jv], neg,
                                           mask=lane0 & actv)
                        cy1 = plsc.load_gather(boxes_v, [biv, k0])
                        cx1 = plsc.load_gather(boxes_v, [biv, k1])
                        cy2 = plsc.load_gather(boxes_v, [biv, k2])
                        cx2 = plsc.load_gather(boxes_v, [biv, k3])
                        acts.append(act)
                        mms.append(mm)
                        bivs.append(biv)
                        cys.append((cy1, cx1, cy2, cx2))

                    areas = [(cy[2] - cy[0]) * (cy[3] - cy[1]) for cy in cys]
                    ioutrips = [jnp.where(a,
                                          lax.shift_right_logical(npk + 31, 5),
                                          0)
                                for a, npk in zip(acts, npks3)]

                    def iou_pair(v, accs):
                        accs = list(accs)
                        for u in range(2):
                            sl = pl.ds((v * 2 + u) * L, L)
                            for pi, P in enumerate(probs):
                                cy1, cx1, cy2, cx2 = cys[pi]
                                p1 = P["py1"][sl]
                                q1 = P["px1"][sl]
                                p2 = P["py2"][sl]
                                q2 = P["px2"][sl]
                                yy1 = jnp.maximum(cy1, p1)
                                xx1 = jnp.maximum(cx1, q1)
                                yy2 = jnp.minimum(cy2, p2)
                                xx2 = jnp.minimum(cx2, q2)
                                inter = (jnp.maximum(zf, yy2 - yy1)
                                         * jnp.maximum(zf, xx2 - xx1))
                                area_p = (p2 - p1) * (q2 - q1)
                                union = (area_p + areas[pi]) - inter
                                t2 = (inter + inter) - union
                                accs[pi] = accs[pi] | (
                                    (union > zf)
                                    & ((t2 * jnp.float32(2.0 ** 24)) > union))
                        return tuple(accs)

                    supvs = lax.fori_loop(0, _maxl(ioutrips), iou_pair,
                                          (jnp.zeros((L,), jnp.bool_),) * nP)

                    out_npks = []
                    for pi, P in enumerate(probs):
                        keep = acts[pi] & jnp.logical_not(jnp.any(supvs[pi]))

                        def _append(P=P, pi=pi, keep=keep):
                            w = jnp.full((L,), npks3[pi], jnp.int32)
                            cy1, cx1, cy2, cx2 = cys[pi]
                            plsc.store_scatter(P["py1"], [w], cy1, mask=lane0)
                            plsc.store_scatter(P["px1"], [w], cx1, mask=lane0)
                            plsc.store_scatter(P["py2"], [w], cy2, mask=lane0)
                            plsc.store_scatter(P["px2"], [w], cx2, mask=lane0)
                            plsc.store_scatter(P["oi"], [w], bivs[pi], mask=lane0)
                            plsc.store_scatter(P["os"], [w], mms[pi], mask=lane0)

                        pl.when(keep)(_append)
                        out_npks.append(npks3[pi] + jnp.where(keep, 1, 0))
                    return tuple(out_npks)

                wtrip = _maxl([jnp.where(lv, cn, 0)
                               for lv, cn in zip(lives, cnts)])
                return lax.fori_loop(0, wtrip, walk_body, npks2)

            return lax.fori_loop(0, 16, bucket_body, npks)

        lax.fori_loop(0, NCH, chunk_body, (jnp.int32(0),) * nP)
        for P, cls in zip(probs, clss):
            pltpu.sync_copy(P["oi"], sh_idx.at[cls])
            pltpu.sync_copy(P["os"], sh_score.at[cls])

    @pl.when(sid >= C - 16)
    def _():
        process_multi([prob0], [sid])

    @pl.when(sid < C - 16)
    def _():
        process_multi([prob0, prob1], [sid, sid + 16])

    plsc.subcore_barrier()

    @pl.when(sid == 0)
    def _():
        pltpu.sync_copy(sh_idx, midx_v)
        pltpu.sync_copy(sh_score, mscore_v)
        zi = jnp.zeros((L,), jnp.int32)
        pos_v[pl.ds(0, L)] = zi
        pos_v[pl.ds(L, L)] = zi

        h0 = plsc.load_gather(mscore_v, [lane, zi])
        cls1 = jnp.minimum(lane + 16, jnp.full((L,), C - 1, jnp.int32))
        h1 = jnp.where(lane < 4, plsc.load_gather(mscore_v, [cls1, zi]), neg)

        def emit_body(slot, st):
            h0c, h1c = st
            m = jnp.maximum(jnp.max(h0c), jnp.max(h1c))
            valid = m > jnp.float32(-jnp.inf)
            validv = jnp.full((L,), valid)
            mm = jnp.full((L,), m)
            cand0 = jnp.where(h0c == mm, lane, bigv)
            cand1 = jnp.where(h1c == mm, lane + 16, bigv)
            cls = jnp.minimum(jnp.min(cand0), jnp.min(cand1))
            cls = jnp.where(valid, cls, 0)
            clsv = jnp.full((L,), cls, jnp.int32)
            p = plsc.load_gather(pos_v, [clsv])
            bi = plsc.load_gather(midx_v, [clsv, p])
            w = jnp.full((L,), slot, jnp.int32)
            plsc.store_scatter(outs_v, [w], jnp.where(validv, mm, -1.0), mask=lane0)
            plsc.store_scatter(outl_v, [w], jnp.where(validv, clsv, -1), mask=lane0)
            plsc.store_scatter(oi0, [w], jnp.where(validv, bi, -1), mask=lane0)
            p2 = p + 1
            newhead = plsc.load_gather(mscore_v, [clsv, p2])
            plsc.store_scatter(pos_v, [clsv], p2, mask=lane0 & validv)
            h0n = jnp.where((lane == clsv) & validv, newhead, h0c)
            h1n = jnp.where((lane + 16 == clsv) & validv, newhead, h1c)
            return h0n, h1n

        lax.fori_loop(0, MAXD, emit_body, (h0, h1))

        def box_body(v, carry):
            sl = pl.ds(v * L, L)
            iv = oi0[sl]
            ok = iv >= 0
            safe = jnp.where(ok, iv, 0)
            m1 = jnp.full((L,), -1.0, jnp.float32)
            outb_v[0, sl] = jnp.where(ok, plsc.load_gather(boxes_v, [safe, k0]), m1)
            outb_v[1, sl] = jnp.where(ok, plsc.load_gather(boxes_v, [safe, k1]), m1)
            outb_v[2, sl] = jnp.where(ok, plsc.load_gather(boxes_v, [safe, k2]), m1)
            outb_v[3, sl] = jnp.where(ok, plsc.load_gather(boxes_v, [safe, k3]), m1)
            return carry

        lax.fori_loop(0, MPAD // L, box_body, 0)

        pltpu.sync_copy(outb_v, outb_hbm.at[b])
        pltpu.sync_copy(outs_v, outs_hbm.at[b])
        pltpu.sync_copy(outl_v, outl_hbm.at[b])


def _prob_scratch():
    return [
        pltpu.VMEM((NPAD,), jnp.float32),
        pltpu.VMEM((NPAD + 4 * L,), jnp.float32),
        pltpu.VMEM((NPAD + 4 * L,), jnp.int32),
        pltpu.VMEM((NPAD + L,), jnp.float32),
        pltpu.VMEM((NPAD + L,), jnp.int32),
        pltpu.VMEM((PPAD,), jnp.float32),
        pltpu.VMEM((PPAD,), jnp.float32),
        pltpu.VMEM((PPAD,), jnp.float32),
        pltpu.VMEM((PPAD,), jnp.float32),
        pltpu.VMEM((MPAD,), jnp.int32),
        pltpu.VMEM((MPAD,), jnp.float32),
    ]


_sc_call = pl.kernel(
    _nms_body,
    out_type=[
        jax.ShapeDtypeStruct((B, 4, MPAD), jnp.float32),
        jax.ShapeDtypeStruct((B, MPAD), jnp.float32),
        jax.ShapeDtypeStruct((B, MPAD), jnp.int32),
    ],
    mesh=plsc.VectorSubcoreMesh(core_axis_name="c", subcore_axis_name="s",
                                num_cores=B, num_subcores=16),
    compiler_params=pltpu.CompilerParams(needs_layout_passes=False,
                                         use_tc_tiling_on_sc=False),
    scratch_types=(
        _prob_scratch() + _prob_scratch() + [
            pltpu.VMEM((N, 4), jnp.float32),
            pltpu.VMEM_SHARED((C, MPAD), jnp.int32),
            pltpu.VMEM_SHARED((C, MPAD), jnp.float32),
            pltpu.VMEM((C, MPAD), jnp.int32),
            pltpu.VMEM((C, MPAD), jnp.float32),
            pltpu.VMEM((2 * L,), jnp.int32),
            pltpu.VMEM((4, MPAD), jnp.float32),
            pltpu.VMEM((MPAD,), jnp.float32),
            pltpu.VMEM((MPAD,), jnp.int32),
        ]
    ),
)


@jax.jit
def kernel(boxes, classification):
    scoresT = jnp.transpose(classification, (0, 2, 1))
    scoresT = jnp.pad(scoresT, ((0, 0), (0, 0), (0, NPAD - N)))
    outb, outs, outl = _sc_call(boxes, scoresT)
    out_boxes = jnp.transpose(outb, (0, 2, 1))[:, :MAXD, :]
    return out_boxes, outs[:, :MAXD], outl[:, :MAXD]


# speedup vs baseline: 1.0198x; 1.0198x over previous
import functools

import jax
import jax.numpy as jnp
from jax import lax
from jax.experimental import pallas as pl
from jax.experimental.pallas import tpu as pltpu
from jax.experimental.pallas import tpu_sc as plsc

N = 5000
NPAD = 5056
C = 20
B = 2
MAXD = 300
MPAD = 304
PPAD = 320
NB = 128
NCH = 8
THR = 0.05
L = 16
NV = NPAD // L
BIGI = 1 << 30


def _maxl(xs):
    out = xs[0]
    for x in xs[1:]:
        out = jnp.maximum(out, x)
    return out


def _nms_body(boxes_hbm, scoresT_hbm, outb_hbm, outs_hbm, outl_hbm,
              sc0, ccs0, cci0, css0, cii0, p10, p20, p30, p40, oi0, os0,
              sc1, ccs1, cci1, css1, cii1, p11, p21, p31, p41, oi1, os1,
              boxes_v, sh_idx, sh_score, midx_v, mscore_v, pos_v,
              outb_v, outs_v, outl_v):
    b = lax.axis_index("c")
    sid = lax.axis_index("s")

    lane = lax.iota(jnp.int32, L)
    lane0 = lane == 0
    zf = jnp.zeros((L,), jnp.float32)
    neg = jnp.full((L,), -jnp.inf, jnp.float32)
    negi1 = jnp.full((L,), -1, jnp.int32)
    bigv = jnp.full((L,), BIGI, jnp.int32)
    thrv = jnp.full((L,), jnp.float32(THR))
    k0 = jnp.zeros((L,), jnp.int32)
    k1 = jnp.full((L,), 1, jnp.int32)
    k2 = jnp.full((L,), 2, jnp.int32)
    k3 = jnp.full((L,), 3, jnp.int32)

    pltpu.sync_copy(boxes_hbm.at[b], boxes_v)

    prob0 = dict(sc=sc0, ccs=ccs0, cci=cci0, css=css0, cii=cii0,
                 py1=p10, px1=p20, py2=p30, px2=p40, oi=oi0, os=os0)
    prob1 = dict(sc=sc1, ccs=ccs1, cci=cci1, css=css1, cii=cii1,
                 py1=p11, px1=p21, py2=p31, px2=p41, oi=oi1, os=os1)

    def process_multi(probs, clss):
        nP = len(probs)
        for P, cls in zip(probs, clss):
            pltpu.sync_copy(scoresT_hbm.at[b, cls], P["sc"])

        def init_body(v, carry):
            sl = pl.ds(v * L, L)
            for P in probs:
                P["py1"][sl] = zf
                P["px1"][sl] = zf
                P["py2"][sl] = zf
                P["px2"][sl] = zf
            return carry

        lax.fori_loop(0, PPAD // L, init_body, 0)

        def inito_body(v, carry):
            sl = pl.ds(v * L, L)
            for P in probs:
                P["oi"][sl] = negi1
                P["os"][sl] = neg
            return carry

        lax.fori_loop(0, MPAD // L, inito_body, 0)

        if nP > 1:
            def memset_body(v, carry):
                sl = pl.ds(v * L, L)
                for P in probs:
                    P["ccs"][sl] = neg
                return carry

            lax.fori_loop(0, (NPAD + 4 * L) // L, memset_body, 0)

            def memset2_body(v, carry):
                sl = pl.ds(v * L, L)
                for P in probs:
                    P["css"][sl] = neg
                return carry

            lax.fori_loop(0, (NPAD + L) // L, memset2_body, 0)

        def chunk_body(cidx, npks):
            actives = [npk < MAXD for npk in npks]
            any_active = actives[0]
            for a in actives[1:]:
                any_active = any_active | a
            activevs = [jnp.full((L,), a) for a in actives]
            klo = (NB - 16) - 16 * cidx
            lo = klo.astype(jnp.float32) * jnp.float32(1.0 / NB)
            hi = jnp.where(cidx == 0, jnp.float32(jnp.inf),
                           (klo + 16).astype(jnp.float32) * jnp.float32(1.0 / NB))
            lov = jnp.full((L,), lo)
            hiv = jnp.full((L,), hi)

            def cscan4(i4, cnts):
                cnts = list(cnts)
                for u in range(4):
                    off = (i4 * 4 + u) * L
                    iv = lane + off
                    for pi, P in enumerate(probs):
                        s = P["sc"][pl.ds(off, L)]
                        g = (s >= lov) & (s < hiv) & (s > thrv) & activevs[pi]
                        csum = plsc.cumsum(g.astype(jnp.int32))
                        pos = (cnts[pi] - 1) + csum
                        plsc.store_scatter(P["ccs"], [pos], s, mask=g)
                        plsc.store_scatter(P["cci"], [pos], iv, mask=g)
                        cnts[pi] = cnts[pi] + jnp.full((L,), csum[15])
                return tuple(cnts)

            ccntvs = lax.fori_loop(0, jnp.where(any_active, NV // 4, 0), cscan4,
                                   (jnp.zeros((L,), jnp.int32),) * nP)
            for pi, P in enumerate(probs):
                for u in range(4):
                    plsc.store_scatter(P["ccs"], [lane + ccntvs[pi] + u * L], neg)
            ccnts = [cv[15] for cv in ccntvs]
            nv4s = [lax.shift_right_logical(cc + 63, 6) for cc in ccnts]

            def bucket_body(b2, npks2):
                k = (klo + 15) - b2
                lives = [npk < MAXD for npk in npks2]
                livevs = [jnp.full((L,), a) for a in lives]
                blo = k.astype(jnp.float32) * jnp.float32(1.0 / NB)
                bhi = jnp.where(k == NB - 1, jnp.float32(jnp.inf),
                                (k + 1).astype(jnp.float32) * jnp.float32(1.0 / NB))
                blov = jnp.full((L,), blo)
                bhiv = jnp.full((L,), bhi)

                def bfilter4(i4, cnts):
                    cnts = list(cnts)
                    for u in range(4):
                        off = (i4 * 4 + u) * L
                        for pi, P in enumerate(probs):
                            s = P["ccs"][pl.ds(off, L)]
                            g = (s >= blov) & (s < bhiv) & livevs[pi]
                            iv = P["cci"][pl.ds(off, L)]
                            csum = plsc.cumsum(g.astype(jnp.int32))
                            pos = (cnts[pi] - 1) + csum
                            plsc.store_scatter(P["css"], [pos], s, mask=g)
                            plsc.store_scatter(P["cii"], [pos], iv, mask=g)
                            cnts[pi] = cnts[pi] + jnp.full((L,), csum[15])
                    return tuple(cnts)

                ftrip = _maxl([jnp.where(lv, n4, 0)
                               for lv, n4 in zip(lives, nv4s)])
                cntvs = lax.fori_loop(0, ftrip, bfilter4,
                                      (jnp.zeros((L,), jnp.int32),) * nP)
                for pi, P in enumerate(probs):
                    plsc.store_scatter(P["css"], [lane + cntvs[pi]], neg)
                cnts = [cv[15] for cv in cntvs]
                nvbs = [lax.shift_right_logical(cn + (L - 1), 4) for cn in cnts]

                lane1 = lane == 1
                lanem4 = lane & jnp.full((L,), 3, jnp.int32)

                def walk_body(t, npks3):
                    lives2 = [npk < MAXD for npk in npks3]
                    mptrips = [jnp.where(lv, nvb, 0)
                               for lv, nvb in zip(lives2, nvbs)]

                    def mp_body(v, st3):
                        st3 = list(st3)
                        posn = lane + v * L
                        for pi, P in enumerate(probs):
                            mv1, pv1, mv2, pv2 = st3[4 * pi:4 * pi + 4]
                            s = P["css"][pl.ds(v * L, L)]
                            upd1 = s > mv1
                            upd2 = jnp.logical_not(upd1) & (s > mv2)
                            st3[4 * pi + 2] = jnp.where(
                                upd1, mv1, jnp.where(upd2, s, mv2))
                            st3[4 * pi + 3] = jnp.where(
                                upd1, pv1, jnp.where(upd2, posn, pv2))
                            st3[4 * pi] = jnp.where(upd1, s, mv1)
                            st3[4 * pi + 1] = jnp.where(upd1, posn, pv1)
                        return tuple(st3)

                    mpst = lax.fori_loop(0, _maxl(mptrips), mp_body,
                                         (neg, bigv, neg, bigv) * nP)

                    acts1, acts2, mm1s, mm2s, biv1s, biv2s, c1s, c2s = (
                        [], [], [], [], [], [], [], [])
                    for pi, P in enumerate(probs):
                        mv1, pv1, mv2, pv2 = mpst[4 * pi:4 * pi + 4]
                        m1 = jnp.max(mv1)
                        act1 = lives2[pi] & (m1 > jnp.float32(-jnp.inf))
                        act1v = jnp.full((L,), act1)
                        mm1 = jnp.full((L,), m1)
                        j1v = jnp.where(
                            act1v,
                            jnp.full((L,), jnp.min(
                                jnp.where(mv1 == mm1, pv1, bigv))),
                            k0)
                        onj1 = pv1 == j1v
                        mvadj = jnp.where(onj1 & act1v, mv2, mv1)
                        padj = jnp.where(onj1 & act1v, pv2, pv1)
                        m2 = jnp.max(mvadj)
                        act2 = act1 & (m2 > jnp.float32(-jnp.inf))
                        act2v = jnp.full((L,), act2)
                        mm2 = jnp.full((L,), m2)
                        j2v = jnp.where(
                            act2v,
                            jnp.full((L,), jnp.min(
                                jnp.where(mvadj == mm2, padj, bigv))),
                            k0)
                        consume_idx = jnp.where(lane0, j1v, j2v)
                        consume_mask = (lane0 & act1v) | (lane1 & act2v)
                        plsc.store_scatter(P["css"], [consume_idx], neg,
                                           mask=consume_mask)
                        biv1 = jnp.where(act1v,
                                         plsc.load_gather(P["cii"], [j1v]), k0)
                        biv2 = jnp.where(act2v,
                                         plsc.load_gather(P["cii"], [j2v]), k0)
                        g1 = plsc.load_gather(boxes_v, [biv1, lanem4])
                        g2 = plsc.load_gather(boxes_v, [biv2, lanem4])
                        c1 = (jnp.full((L,), g1[0]), jnp.full((L,), g1[1]),
                              jnp.full((L,), g1[2]), jnp.full((L,), g1[3]))
                        c2 = (jnp.full((L,), g2[0]), jnp.full((L,), g2[1]),
                              jnp.full((L,), g2[2]), jnp.full((L,), g2[3]))
                        acts1.append(act1)
                        acts2.append(act2)
                        mm1s.append(mm1)
                        mm2s.append(mm2)
                        biv1s.append(biv1)
                        biv2s.append(biv2)
                        c1s.append(c1)
                        c2s.append(c2)

                    a1s = [(c[2] - c[0]) * (c[3] - c[1]) for c in c1s]
                    a2s = [(c[2] - c[0]) * (c[3] - c[1]) for c in c2s]
                    ioutrips = [jnp.where(a,
                                          lax.shift_right_logical(npk + 31, 5),
                                          0)
                                for a, npk in zip(acts1, npks3)]

                    def iou_pair(v, accs):
                        accs = list(accs)
                        for u in range(2):
                            sl = pl.ds((v * 2 + u) * L, L)
                            for pi, P in enumerate(probs):
                                p1 = P["py1"][sl]
                                q1 = P["px1"][sl]
                                p2 = P["py2"][sl]
                                q2 = P["px2"][sl]
                                area_p = (p2 - p1) * (q2 - q1)
                                for ci, (cand, ac) in enumerate(
                                        ((c1s[pi], a1s[pi]),
                                         (c2s[pi], a2s[pi]))):
                                    cy1, cx1, cy2, cx2 = cand
                                    yy1 = jnp.maximum(cy1, p1)
                                    xx1 = jnp.maximum(cx1, q1)
                                    yy2 = jnp.minimum(cy2, p2)
                                    xx2 = jnp.minimum(cx2, q2)
                                    inter = (jnp.maximum(zf, yy2 - yy1)
                                             * jnp.maximum(zf, xx2 - xx1))
                                    union = (area_p + ac) - inter
                                    t2 = (inter + inter) - union
                                    accs[2 * pi + ci] = accs[2 * pi + ci] | (
                                        (union > zf)
                                        & ((t2 * jnp.float32(2.0 ** 24))
                                           > union))
                        return tuple(accs)

                    supvs = lax.fori_loop(0, _maxl(ioutrips), iou_pair,
                                          (jnp.zeros((L,), jnp.bool_),)
                                          * (2 * nP))

                    out_npks = []
                    for pi, P in enumerate(probs):
                        keep1 = acts1[pi] & jnp.logical_not(
                            jnp.any(supvs[2 * pi]))
                        keep1v = jnp.full((L,), keep1)
                        cy1, cx1, cy2, cx2 = c2s[pi]
                        p1, q1, p2, q2 = c1s[pi]
                        yy1 = jnp.maximum(cy1, p1)
                        xx1 = jnp.maximum(cx1, q1)
                        yy2 = jnp.minimum(cy2, p2)
                        xx2 = jnp.minimum(cx2, q2)
                        inter = (jnp.maximum(zf, yy2 - yy1)
                                 * jnp.maximum(zf, xx2 - xx1))
                        union = (a1s[pi] + a2s[pi]) - inter
                        t2 = (inter + inter) - union
                        sup12 = (union > zf) & (
                            (t2 * jnp.float32(2.0 ** 24)) > union)
                        sup2all = supvs[2 * pi + 1] | (keep1v & sup12)
                        keep1i = jnp.where(keep1, 1, 0)
                        keep2 = (acts2[pi]
                                 & jnp.logical_not(jnp.any(sup2all))
                                 & ((npks3[pi] + keep1i) < MAXD))

                        def _append1(P=P, pi=pi):
                            w = jnp.full((L,), npks3[pi], jnp.int32)
                            cy1, cx1, cy2, cx2 = c1s[pi]
                            plsc.store_scatter(P["py1"], [w], cy1, mask=lane0)
                            plsc.store_scatter(P["px1"], [w], cx1, mask=lane0)
                            plsc.store_scatter(P["py2"], [w], cy2, mask=lane0)
                            plsc.store_scatter(P["px2"], [w], cx2, mask=lane0)
                            plsc.store_scatter(P["oi"], [w], biv1s[pi],
                                               mask=lane0)
                            plsc.store_scatter(P["os"], [w], mm1s[pi],
                                               mask=lane0)

                        pl.when(keep1)(_append1)

                        def _append2(P=P, pi=pi, keep1i=keep1i):
                            w = jnp.full((L,), npks3[pi] + keep1i, jnp.int32)
                            cy1, cx1, cy2, cx2 = c2s[pi]
                            plsc.store_scatter(P["py1"], [w], cy1, mask=lane0)
                            plsc.store_scatter(P["px1"], [w], cx1, mask=lane0)
                            plsc.store_scatter(P["py2"], [w], cy2, mask=lane0)
                            plsc.store_scatter(P["px2"], [w], cx2, mask=lane0)
                            plsc.store_scatter(P["oi"], [w], biv2s[pi],
                                               mask=lane0)
                            plsc.store_scatter(P["os"], [w], mm2s[pi],
                                               mask=lane0)

                        pl.when(keep2)(_append2)
                        out_npks.append(npks3[pi] + keep1i
                                        + jnp.where(keep2, 1, 0))
                    return tuple(out_npks)

                wtrip = _maxl([jnp.where(lv,
                                         lax.shift_right_logical(cn + 1, 1),
                                         0)
                               for lv, cn in zip(lives, cnts)])
                return lax.fori_loop(0, wtrip, walk_body, npks2)

            return lax.fori_loop(0, 16, bucket_body, npks)

        lax.fori_loop(0, NCH, chunk_body, (jnp.int32(0),) * nP)
        for P, cls in zip(probs, clss):
            pltpu.sync_copy(P["oi"], sh_idx.at[cls])
            pltpu.sync_copy(P["os"], sh_score.at[cls])

    @pl.when(sid >= C - 16)
    def _():
        process_multi([prob0], [sid])

    @pl.when(sid < C - 16)
    def _():
        process_multi([prob0, prob1], [sid, sid + 16])

    plsc.subcore_barrier()

    @pl.when(sid == 0)
    def _():
        pltpu.sync_copy(sh_idx, midx_v)
        pltpu.sync_copy(sh_score, mscore_v)
        zi = jnp.zeros((L,), jnp.int32)
        pos_v[pl.ds(0, L)] = zi
        pos_v[pl.ds(L, L)] = zi

        h0 = plsc.load_gather(mscore_v, [lane, zi])
        cls1 = jnp.minimum(lane + 16, jnp.full((L,), C - 1, jnp.int32))
        h1 = jnp.where(lane < 4, plsc.load_gather(mscore_v, [cls1, zi]), neg)

        def emit_body(slot, st):
            h0c, h1c = st
            m = jnp.maximum(jnp.max(h0c), jnp.max(h1c))
            valid = m > jnp.float32(-jnp.inf)
            validv = jnp.full((L,), valid)
            mm = jnp.full((L,), m)
            cand0 = jnp.where(h0c == mm, lane, bigv)
            cand1 = jnp.where(h1c == mm, lane + 16, bigv)
            cls = jnp.minimum(jnp.min(cand0), jnp.min(cand1))
            cls = jnp.where(valid, cls, 0)
            clsv = jnp.full((L,), cls, jnp.int32)
            p = plsc.load_gather(pos_v, [clsv])
            bi = plsc.load_gather(midx_v, [clsv, p])
            w = jnp.full((L,), slot, jnp.int32)
            plsc.store_scatter(outs_v, [w], jnp.where(validv, mm, -1.0), mask=lane0)
            plsc.store_scatter(outl_v, [w], jnp.where(validv, clsv, -1), mask=lane0)
            plsc.store_scatter(oi0, [w], jnp.where(validv, bi, -1), mask=lane0)
            p2 = p + 1
            newhead = plsc.load_gather(mscore_v, [clsv, p2])
            plsc.store_scatter(pos_v, [clsv], p2, mask=lane0 & validv)
            h0n = jnp.where((lane == clsv) & validv, newhead, h0c)
            h1n = jnp.where((lane + 16 == clsv) & validv, newhead, h1c)
            return h0n, h1n

        lax.fori_loop(0, MAXD, emit_body, (h0, h1))

        def box_body(v, carry):
            sl = pl.ds(v * L, L)
            iv = oi0[sl]
            ok = iv >= 0
            safe = jnp.where(ok, iv, 0)
            m1 = jnp.full((L,), -1.0, jnp.float32)
            outb_v[0, sl] = jnp.where(ok, plsc.load_gather(boxes_v, [safe, k0]), m1)
            outb_v[1, sl] = jnp.where(ok, plsc.load_gather(boxes_v, [safe, k1]), m1)
            outb_v[2, sl] = jnp.where(ok, plsc.load_gather(boxes_v, [safe, k2]), m1)
            outb_v[3, sl] = jnp.where(ok, plsc.load_gather(boxes_v, [safe, k3]), m1)
            return carry

        lax.fori_loop(0, MPAD // L, box_body, 0)

        pltpu.sync_copy(outb_v, outb_hbm.at[b])
        pltpu.sync_copy(outs_v, outs_hbm.at[b])
        pltpu.sync_copy(outl_v, outl_hbm.at[b])


def _prob_scratch():
    return [
        pltpu.VMEM((NPAD,), jnp.float32),
        pltpu.VMEM((NPAD + 4 * L,), jnp.float32),
        pltpu.VMEM((NPAD + 4 * L,), jnp.int32),
        pltpu.VMEM((NPAD + L,), jnp.float32),
        pltpu.VMEM((NPAD + L,), jnp.int32),
        pltpu.VMEM((PPAD,), jnp.float32),
        pltpu.VMEM((PPAD,), jnp.float32),
        pltpu.VMEM((PPAD,), jnp.float32),
        pltpu.VMEM((PPAD,), jnp.float32),
        pltpu.VMEM((MPAD,), jnp.int32),
        pltpu.VMEM((MPAD,), jnp.float32),
    ]


_sc_call = pl.kernel(
    _nms_body,
    out_type=[
        jax.ShapeDtypeStruct((B, 4, MPAD), jnp.float32),
        jax.ShapeDtypeStruct((B, MPAD), jnp.float32),
        jax.ShapeDtypeStruct((B, MPAD), jnp.int32),
    ],
    mesh=plsc.VectorSubcoreMesh(core_axis_name="c", subcore_axis_name="s",
                                num_cores=B, num_subcores=16),
    compiler_params=pltpu.CompilerParams(needs_layout_passes=False,
                                         use_tc_tiling_on_sc=False),
    scratch_types=(
        _prob_scratch() + _prob_scratch() + [
            pltpu.VMEM((N, 4), jnp.float32),
            pltpu.VMEM_SHARED((C, MPAD), jnp.int32),
            pltpu.VMEM_SHARED((C, MPAD), jnp.float32),
            pltpu.VMEM((C, MPAD), jnp.int32),
            pltpu.VMEM((C, MPAD), jnp.float32),
            pltpu.VMEM((2 * L,), jnp.int32),
            pltpu.VMEM((4, MPAD), jnp.float32),
            pltpu.VMEM((MPAD,), jnp.float32),
            pltpu.VMEM((MPAD,), jnp.int32),
        ]
    ),
)


@jax.jit
def kernel(boxes, classification):
    scoresT = jnp.transpose(classification, (0, 2, 1))
    scoresT = jnp.pad(scoresT, ((0, 0), (0, 0), (0, NPAD - N)))
    outb, outs, outl = _sc_call(boxes, scoresT)
    out_boxes = jnp.transpose(outb, (0, 2, 1))[:, :MAXD, :]
    return out_boxes, outs[:, :MAXD], outl[:, :MAXD]
